# baseline (device time: 134037 ns/iter reference)
import jax
import jax.numpy as jnp
from jax import lax
from jax.experimental import pallas as pl
from jax.experimental.pallas import tpu as pltpu

N_DEV = 8
M = 1536
D = 1536
H = 3072
BH = 384
NK = H // BH

NG = 3
GCOLS = D // NG
DIM_ORDERS = ((1, 3, 4), (3, 4, 1), (4, 1, 3))


def _ar_tail(p_out, gb, bufs, sems, i):
    b0 = i & 1
    b1 = (i >> 1) & 1
    b2 = (i >> 2) & 1
    par = (i ^ (i >> 1)) & 1
    roles = ((par, b0, b2), (b1, b2, b0), (b2, par, b1))

    barrier = pltpu.get_barrier_semaphore()
    for m in (1, 3, 4):
        pl.semaphore_signal(barrier, inc=1, device_id=(i ^ m,),
                            device_id_type=pl.DeviceIdType.MESH)
    pl.semaphore_wait(barrier, 3)

    def xchg(src, dst, send_sem, recv_sem, mask):
        rd = pltpu.make_async_remote_copy(
            src_ref=src, dst_ref=dst, send_sem=send_sem, recv_sem=recv_sem,
            device_id=(i ^ mask,), device_id_type=pl.DeviceIdType.MESH)
        rd.start()
        return rd

    bf16 = jnp.bfloat16
    f32 = jnp.float32
    co = [g * GCOLS for g in range(NG)]

    rds = []
    for g in range(NG):
        snd1, _, _, rcv1 = bufs[g][0], bufs[g][1], bufs[g][2], bufs[g][3]
        h = roles[g][0]
        snd1[...] = p_out[pl.ds((1 - h) * 768, 768),
                          pl.ds(co[g], GCOLS)].astype(bf16)
        rds.append(xchg(snd1, rcv1, sems[g][0].at[0], sems[g][1].at[0],
                        DIM_ORDERS[g][0]))
    for g in range(NG):
        rds[g].wait()
        rcv1, acc1 = bufs[g][3], bufs[g][6]
        h = roles[g][0]
        acc1[...] = (rcv1[...].astype(f32)
                     + p_out[pl.ds(h * 768, 768), pl.ds(co[g], GCOLS)])

    rds = []
    for g in range(NG):
        snd2, rcv2, acc1 = bufs[g][1], bufs[g][4], bufs[g][6]
        q = roles[g][1]
        snd2[...] = acc1[pl.ds((1 - q) * 384, 384), :].astype(bf16)
        rds.append(xchg(snd2, rcv2, sems[g][0].at[1], sems[g][1].at[1],
                        DIM_ORDERS[g][1]))
    for g in range(NG):
        rds[g].wait()
        rcv2, acc1, acc2 = bufs[g][4], bufs[g][6], bufs[g][7]
        q = roles[g][1]
        acc2[...] = rcv2[...].astype(f32) + acc1[pl.ds(q * 384, 384), :]

    rds = []
    for g in range(NG):
        snd3, rcv3, acc2 = bufs[g][2], bufs[g][5], bufs[g][7]
        r = roles[g][2]
        snd3[...] = acc2[pl.ds((1 - r) * 192, 192), :].astype(bf16)
        rds.append(xchg(snd3, rcv3, sems[g][0].at[2], sems[g][1].at[2],
                        DIM_ORDERS[g][2]))
    own = []
    for g in range(NG):
        rds[g].wait()
        rcv3, acc2 = bufs[g][5], bufs[g][7]
        h, q, r = roles[g]
        c = 4 * h + 2 * q + r
        own.append(c)
        acc3 = rcv3[...].astype(f32) + acc2[pl.ds(r * 192, 192), :]
        p_out[pl.ds(c * 192, 192), pl.ds(co[g], GCOLS)] = acc3
        gb[pl.ds(c * 192, 192), pl.ds(co[g], GCOLS)] = acc3.astype(bf16)

    rds = []
    for g in range(NG):
        seg = own[g] * 192
        sl = (pl.ds(seg, 192), pl.ds(co[g], GCOLS))
        rds.append(xchg(gb.at[sl[0], sl[1]], gb.at[sl[0], sl[1]],
                        sems[g][2].at[0], sems[g][3].at[0],
                        DIM_ORDERS[g][2]))
    for g in range(NG):
        rds[g].wait()
    rds = []
    for g in range(NG):
        h, q, r = roles[g]
        seg = (4 * h + 2 * q) * 192
        rds.append(xchg(gb.at[pl.ds(seg, 384), pl.ds(co[g], GCOLS)],
                        gb.at[pl.ds(seg, 384), pl.ds(co[g], GCOLS)],
                        sems[g][2].at[1], sems[g][3].at[1],
                        DIM_ORDERS[g][1]))
    for g in range(NG):
        h, q, r = roles[g]
        p1 = (4 * h + 2 * q + 1 - r) * 192
        p_out[pl.ds(p1, 192), pl.ds(co[g], GCOLS)] = (
            gb[pl.ds(p1, 192), pl.ds(co[g], GCOLS)].astype(f32))
    for g in range(NG):
        rds[g].wait()
    rds = []
    for g in range(NG):
        h = roles[g][0]
        rds.append(xchg(gb.at[pl.ds(h * 768, 768), pl.ds(co[g], GCOLS)],
                        gb.at[pl.ds(h * 768, 768), pl.ds(co[g], GCOLS)],
                        sems[g][2].at[2], sems[g][3].at[2],
                        DIM_ORDERS[g][0]))
    for g in range(NG):
        h, q, r = roles[g]
        p2 = (4 * h + 2 * (1 - q)) * 192
        p_out[pl.ds(p2, 384), pl.ds(co[g], GCOLS)] = (
            gb[pl.ds(p2, 384), pl.ds(co[g], GCOLS)].astype(f32))
    for g in range(NG):
        rds[g].wait()
    for g in range(NG):
        h = roles[g][0]
        p_out[pl.ds((1 - h) * 768, 768), pl.ds(co[g], GCOLS)] = (
            gb[pl.ds((1 - h) * 768, 768), pl.ds(co[g], GCOLS)].astype(f32))


def _fused_body(x_ref, wg_ref, wu_ref, wd_ref, out_ref, x16_ref, *refs):
    bufs = [refs[8 * g:8 * (g + 1)] for g in range(NG)]
    sems = [refs[8 * NG + 4 * g:8 * NG + 4 * (g + 1)] for g in range(NG)]
    k = pl.program_id(0)

    @pl.when(k == 0)
    def _():
        x16_ref[...] = x_ref[...].astype(jnp.bfloat16)

    x16 = x16_ref[...]
    g = jnp.dot(x16, wg_ref[...].astype(jnp.bfloat16),
                preferred_element_type=jnp.float32)
    u = jnp.dot(x16, wu_ref[...].astype(jnp.bfloat16),
                preferred_element_type=jnp.float32)
    a = g * (u * jax.nn.sigmoid(u))
    part = jnp.dot(a.astype(jnp.bfloat16), wd_ref[...].astype(jnp.bfloat16),
                   preferred_element_type=jnp.float32)

    @pl.when(k == 0)
    def _():
        out_ref[...] = part

    @pl.when(k > 0)
    def _():
        out_ref[...] += part

    @pl.when(k == NK - 1)
    def _():
        _ar_tail(out_ref, x16_ref, bufs, sems, lax.axis_index("i"))


def kernel(x, Wg, Wu, Wd):
    group_bufs = [
        pltpu.VMEM((768, GCOLS), jnp.bfloat16),
        pltpu.VMEM((384, GCOLS), jnp.bfloat16),
        pltpu.VMEM((192, GCOLS), jnp.bfloat16),
        pltpu.VMEM((768, GCOLS), jnp.bfloat16),
        pltpu.VMEM((384, GCOLS), jnp.bfloat16),
        pltpu.VMEM((192, GCOLS), jnp.bfloat16),
        pltpu.VMEM((768, GCOLS), jnp.float32),
        pltpu.VMEM((384, GCOLS), jnp.float32),
    ]
    sems = pltpu.SemaphoreType.DMA((3,))
    return pl.pallas_call(
        _fused_body,
        grid=(NK,),
        in_specs=[
            pl.BlockSpec(memory_space=pltpu.VMEM),
            pl.BlockSpec((D, BH), lambda k: (0, k)),
            pl.BlockSpec((D, BH), lambda k: (0, k)),
            pl.BlockSpec((BH, D), lambda k: (k, 0)),
        ],
        out_specs=pl.BlockSpec(memory_space=pltpu.VMEM),
        out_shape=jax.ShapeDtypeStruct((M, D), jnp.float32),
        scratch_shapes=(
            [pltpu.VMEM((M, D), jnp.bfloat16)]
            + group_bufs * NG
            + [sems] * (4 * NG)
        ),
        compiler_params=pltpu.CompilerParams(
            collective_id=0, vmem_limit_bytes=63 * 1024 * 1024),
    )(x, Wg, Wu, Wd)


# device time: 118870 ns/iter; 1.1276x vs baseline; 1.1276x over previous
import jax
import jax.numpy as jnp
from jax import lax
from jax.experimental import pallas as pl
from jax.experimental.pallas import tpu as pltpu

N_DEV = 8
M = 1536
D = 1536
H = 3072
BH = 512
NK = H // BH

NG = 3
GCOLS = D // NG
DIM_ORDERS = ((1, 3, 4), (3, 4, 1), (4, 1, 3))


def _ar_tail(p_out, gb, bufs, sems, i):
    b0 = i & 1
    b1 = (i >> 1) & 1
    b2 = (i >> 2) & 1
    par = (i ^ (i >> 1)) & 1
    roles = ((par, b0, b2), (b1, b2, b0), (b2, par, b1))

    barrier = pltpu.get_barrier_semaphore()
    for m in (1, 3, 4):
        pl.semaphore_signal(barrier, inc=1, device_id=(i ^ m,),
                            device_id_type=pl.DeviceIdType.MESH)
    pl.semaphore_wait(barrier, 3)

    def xchg(src, dst, send_sem, recv_sem, mask):
        rd = pltpu.make_async_remote_copy(
            src_ref=src, dst_ref=dst, send_sem=send_sem, recv_sem=recv_sem,
            device_id=(i ^ mask,), device_id_type=pl.DeviceIdType.MESH)
        rd.start()
        return rd

    bf16 = jnp.bfloat16
    f32 = jnp.float32
    co = [g * GCOLS for g in range(NG)]

    rds = []
    for g in range(NG):
        h = roles[g][0]
        gb[pl.ds(0, 768), pl.ds(co[g], GCOLS)] = (
            p_out[pl.ds((1 - h) * 768, 768), pl.ds(co[g], GCOLS)].astype(bf16))
        rds.append(xchg(gb.at[pl.ds(0, 768), pl.ds(co[g], GCOLS)],
                        bufs[g][1], sems[g][0].at[0], sems[g][1].at[0],
                        DIM_ORDERS[g][0]))
    for g in range(NG):
        rds[g].wait()

    rds = []
    for g in range(NG):
        rcv1 = bufs[g][1]
        h, q, _ = roles[g]
        gb[pl.ds(768, 384), pl.ds(co[g], GCOLS)] = (
            rcv1[pl.ds((1 - q) * 384, 384), :].astype(f32)
            + p_out[pl.ds(h * 768 + (1 - q) * 384, 384), pl.ds(co[g], GCOLS)]
        ).astype(bf16)
        rds.append(xchg(gb.at[pl.ds(768, 384), pl.ds(co[g], GCOLS)],
                        bufs[g][2], sems[g][0].at[1], sems[g][1].at[1],
                        DIM_ORDERS[g][1]))
    for g in range(NG):
        rds[g].wait()
        rcv1, rcv2, acc2 = bufs[g][1], bufs[g][2], bufs[g][4]
        h, q, _ = roles[g]
        acc2[...] = (rcv2[...].astype(f32)
                     + rcv1[pl.ds(q * 384, 384), :].astype(f32)
                     + p_out[pl.ds(h * 768 + q * 384, 384),
                             pl.ds(co[g], GCOLS)])

    rds = []
    for g in range(NG):
        snd3, rcv3, acc2 = bufs[g][0], bufs[g][3], bufs[g][4]
        r = roles[g][2]
        snd3[...] = acc2[pl.ds((1 - r) * 192, 192), :].astype(bf16)
        rds.append(xchg(snd3, rcv3, sems[g][0].at[2], sems[g][1].at[2],
                        DIM_ORDERS[g][2]))
    own = []
    for g in range(NG):
        rds[g].wait()
        rcv3, acc2 = bufs[g][3], bufs[g][4]
        h, q, r = roles[g]
        c = 4 * h + 2 * q + r
        own.append(c)
        acc3 = rcv3[...].astype(f32) + acc2[pl.ds(r * 192, 192), :]
        p_out[pl.ds(c * 192, 192), pl.ds(co[g], GCOLS)] = acc3
        gb[pl.ds(c * 192, 192), pl.ds(co[g], GCOLS)] = acc3.astype(bf16)

    rds = []
    for g in range(NG):
        seg = own[g] * 192
        sl = (pl.ds(seg, 192), pl.ds(co[g], GCOLS))
        rds.append(xchg(gb.at[sl[0], sl[1]], gb.at[sl[0], sl[1]],
                        sems[g][2].at[0], sems[g][3].at[0],
                        DIM_ORDERS[g][2]))
    for g in range(NG):
        rds[g].wait()
    rds = []
    for g in range(NG):
        h, q, r = roles[g]
        seg = (4 * h + 2 * q) * 192
        rds.append(xchg(gb.at[pl.ds(seg, 384), pl.ds(co[g], GCOLS)],
                        gb.at[pl.ds(seg, 384), pl.ds(co[g], GCOLS)],
                        sems[g][2].at[1], sems[g][3].at[1],
                        DIM_ORDERS[g][1]))
    for g in range(NG):
        h, q, r = roles[g]
        p1 = (4 * h + 2 * q + 1 - r) * 192
        p_out[pl.ds(p1, 192), pl.ds(co[g], GCOLS)] = (
            gb[pl.ds(p1, 192), pl.ds(co[g], GCOLS)].astype(f32))
    for g in range(NG):
        rds[g].wait()
    rds = []
    for g in range(NG):
        h = roles[g][0]
        rds.append(xchg(gb.at[pl.ds(h * 768, 768), pl.ds(co[g], GCOLS)],
                        gb.at[pl.ds(h * 768, 768), pl.ds(co[g], GCOLS)],
                        sems[g][2].at[2], sems[g][3].at[2],
                        DIM_ORDERS[g][0]))
    for g in range(NG):
        h, q, r = roles[g]
        p2 = (4 * h + 2 * (1 - q)) * 192
        p_out[pl.ds(p2, 384), pl.ds(co[g], GCOLS)] = (
            gb[pl.ds(p2, 384), pl.ds(co[g], GCOLS)].astype(f32))
    for g in range(NG):
        rds[g].wait()
    for g in range(NG):
        h = roles[g][0]
        p_out[pl.ds((1 - h) * 768, 768), pl.ds(co[g], GCOLS)] = (
            gb[pl.ds((1 - h) * 768, 768), pl.ds(co[g], GCOLS)].astype(f32))


def _fused_body(x_ref, wg_ref, wu_ref, wd_ref, out_ref, x16_ref, *refs):
    bufs = [refs[5 * g:5 * (g + 1)] for g in range(NG)]
    sems = [refs[5 * NG + 4 * g:5 * NG + 4 * (g + 1)] for g in range(NG)]
    k = pl.program_id(0)

    @pl.when(k == 0)
    def _():
        x16_ref[...] = x_ref[...].astype(jnp.bfloat16)

    x16 = x16_ref[...]
    g = jnp.dot(x16, wg_ref[...].astype(jnp.bfloat16),
                preferred_element_type=jnp.float32)
    u = jnp.dot(x16, wu_ref[...].astype(jnp.bfloat16),
                preferred_element_type=jnp.float32)
    a = g * (u * jax.nn.sigmoid(u))
    part = jnp.dot(a.astype(jnp.bfloat16), wd_ref[...].astype(jnp.bfloat16),
                   preferred_element_type=jnp.float32)

    @pl.when(k == 0)
    def _():
        out_ref[...] = part

    @pl.when(k > 0)
    def _():
        out_ref[...] += part

    @pl.when(k == NK - 1)
    def _():
        _ar_tail(out_ref, x16_ref, bufs, sems, lax.axis_index("i"))


def kernel(x, Wg, Wu, Wd):
    group_bufs = [
        pltpu.VMEM((192, GCOLS), jnp.bfloat16),
        pltpu.VMEM((768, GCOLS), jnp.bfloat16),
        pltpu.VMEM((384, GCOLS), jnp.bfloat16),
        pltpu.VMEM((192, GCOLS), jnp.bfloat16),
        pltpu.VMEM((384, GCOLS), jnp.float32),
    ]
    sems = pltpu.SemaphoreType.DMA((3,))
    return pl.pallas_call(
        _fused_body,
        grid=(NK,),
        in_specs=[
            pl.BlockSpec(memory_space=pltpu.VMEM),
            pl.BlockSpec((D, BH), lambda k: (0, k)),
            pl.BlockSpec((D, BH), lambda k: (0, k)),
            pl.BlockSpec((BH, D), lambda k: (k, 0)),
        ],
        out_specs=pl.BlockSpec(memory_space=pltpu.VMEM),
        out_shape=jax.ShapeDtypeStruct((M, D), jnp.float32),
        scratch_shapes=(
            [pltpu.VMEM((M, D), jnp.bfloat16)]
            + group_bufs * NG
            + [sems] * (4 * NG)
        ),
        compiler_params=pltpu.CompilerParams(
            collective_id=0, vmem_limit_bytes=63 * 1024 * 1024),
    )(x, Wg, Wu, Wd)


# device time: 110486 ns/iter; 1.2132x vs baseline; 1.0759x over previous
import jax
import jax.numpy as jnp
from jax import lax
from jax.experimental import pallas as pl
from jax.experimental.pallas import tpu as pltpu

N_DEV = 8
M = 1536
D = 1536
H = 3072
BH = 512
NK = H // BH

NG = 3
GCOLS = D // NG
DIM_ORDERS = ((1, 3, 4), (3, 4, 1), (4, 1, 3))
HR = M // 2
S1, S2, S3 = HR // 2, HR // 4, HR // 8


def _roles(i):
    b0 = i & 1
    b1 = (i >> 1) & 1
    b2 = (i >> 2) & 1
    par = (i ^ (i >> 1)) & 1
    return ((par, b0, b2), (b1, b2, b0), (b2, par, b1))


def _ar_phase(t, base, p_out, gb, bufs, sems, i):
    roles = _roles(i)
    bf16 = jnp.bfloat16
    f32 = jnp.float32
    co = [g * GCOLS for g in range(NG)]

    def xchg(src, dst, send_sem, recv_sem, mask, start):
        rd = pltpu.make_async_remote_copy(
            src_ref=src, dst_ref=dst, send_sem=send_sem, recv_sem=recv_sem,
            device_id=(i ^ mask,), device_id_type=pl.DeviceIdType.MESH)
        if start:
            rd.start()
        return rd

    def rs_rd(g, n, start):
        h, q, r = roles[g]
        if n == 0:
            src = gb.at[pl.ds(base, S1), pl.ds(co[g], GCOLS)]
            dst, size = bufs[g][1], S1
        elif n == 1:
            src = gb.at[pl.ds(base + S1, S2), pl.ds(co[g], GCOLS)]
            dst, size = bufs[g][2], S2
        else:
            src = bufs[g][0]
            dst, size = bufs[g][3], S3
        return xchg(src, dst, sems[g][0].at[n], sems[g][1].at[n],
                    DIM_ORDERS[g][n], start)

    def ag_seg(g, n):
        h, q, r = roles[g]
        if n == 0:
            return base + (4 * h + 2 * q + r) * S3, S3
        if n == 1:
            return base + (4 * h + 2 * q) * S3, 2 * S3
        return base + h * S1, S1

    def ag_rd(g, n, start):
        seg, sz = ag_seg(g, n)
        ref = gb.at[pl.ds(seg, sz), pl.ds(co[g], GCOLS)]
        return xchg(ref, ref, sems[g][2].at[n], sems[g][3].at[n],
                    DIM_ORDERS[g][2 - n], start)

    if t == 0:
        barrier = pltpu.get_barrier_semaphore()
        for m in (1, 3, 4):
            pl.semaphore_signal(barrier, inc=1, device_id=(i ^ m,),
                                device_id_type=pl.DeviceIdType.MESH)
        pl.semaphore_wait(barrier, 3)
        for g in range(NG):
            h = roles[g][0]
            gb[pl.ds(base, S1), pl.ds(co[g], GCOLS)] = (
                p_out[pl.ds(base + (1 - h) * S1, S1),
                      pl.ds(co[g], GCOLS)].astype(bf16))
        for g in range(NG):
            rs_rd(g, 0, True)
    elif t == 1:
        for g in range(NG):
            rs_rd(g, 0, False).wait()
        for g in range(NG):
            rcv1 = bufs[g][1]
            h, q, _ = roles[g]
            gb[pl.ds(base + S1, S2), pl.ds(co[g], GCOLS)] = (
                rcv1[pl.ds((1 - q) * S2, S2), :].astype(f32)
                + p_out[pl.ds(base + h * S1 + (1 - q) * S2, S2),
                        pl.ds(co[g], GCOLS)]
            ).astype(bf16)
        for g in range(NG):
            rs_rd(g, 1, True)
    elif t == 2:
        for g in range(NG):
            rs_rd(g, 1, False).wait()
        for g in range(NG):
            rcv1, rcv2, acc2 = bufs[g][1], bufs[g][2], bufs[g][4]
            h, q, _ = roles[g]
            acc2[...] = (rcv2[...].astype(f32)
                         + rcv1[pl.ds(q * S2, S2), :].astype(f32)
                         + p_out[pl.ds(base + h * S1 + q * S2, S2),
                                 pl.ds(co[g], GCOLS)])
        for g in range(NG):
            snd3, acc2 = bufs[g][0], bufs[g][4]
            r = roles[g][2]
            snd3[...] = acc2[pl.ds((1 - r) * S3, S3), :].astype(bf16)
        for g in range(NG):
            rs_rd(g, 2, True)
    elif t == 3:
        for g in range(NG):
            rs_rd(g, 2, False).wait()
        for g in range(NG):
            rcv3, acc2 = bufs[g][3], bufs[g][4]
            h, q, r = roles[g]
            c = 4 * h + 2 * q + r
            acc3 = rcv3[...].astype(f32) + acc2[pl.ds(r * S3, S3), :]
            p_out[pl.ds(base + c * S3, S3), pl.ds(co[g], GCOLS)] = acc3
            gb[pl.ds(base + c * S3, S3), pl.ds(co[g], GCOLS)] = (
                acc3.astype(bf16))
        for g in range(NG):
            ag_rd(g, 0, True)
    elif t == 4:
        for g in range(NG):
            ag_rd(g, 0, False).wait()
        for g in range(NG):
            ag_rd(g, 1, True)
    elif t == 5:
        for g in range(NG):
            ag_rd(g, 1, False).wait()
        for g in range(NG):
            h, q, r = roles[g]
            p1 = base + (4 * h + 2 * q + 1 - r) * S3
            p_out[pl.ds(p1, S3), pl.ds(co[g], GCOLS)] = (
                gb[pl.ds(p1, S3), pl.ds(co[g], GCOLS)].astype(f32))
        for g in range(NG):
            ag_rd(g, 2, True)
    else:
        for g in range(NG):
            ag_rd(g, 2, False).wait()
        for g in range(NG):
            h, q, r = roles[g]
            p2 = base + (4 * h + 2 * (1 - q)) * S3
            p_out[pl.ds(p2, 2 * S3), pl.ds(co[g], GCOLS)] = (
                gb[pl.ds(p2, 2 * S3), pl.ds(co[g], GCOLS)].astype(f32))
        for g in range(NG):
            h = roles[g][0]
            p_out[pl.ds(base + (1 - h) * S1, S1), pl.ds(co[g], GCOLS)] = (
                gb[pl.ds(base + (1 - h) * S1, S1),
                   pl.ds(co[g], GCOLS)].astype(f32))


def _fused_body(x_ref, wg_ref, wu_ref, wd_ref, out_ref, x16_ref, *refs):
    bufs = [refs[5 * g:5 * (g + 1)] for g in range(NG)]
    sems = [refs[5 * NG + 4 * g:5 * NG + 4 * (g + 1)] for g in range(NG)]
    k = pl.program_id(0)
    kk = lax.rem(k, NK)
    row0 = (k // NK) * HR
    i = lax.axis_index("i")

    @pl.when(k == 0)
    def _():
        x16_ref[...] = x_ref[...].astype(jnp.bfloat16)

    x16h = x16_ref[pl.ds(row0, HR), :]
    g = jnp.dot(x16h, wg_ref[...].astype(jnp.bfloat16),
                preferred_element_type=jnp.float32)
    u = jnp.dot(x16h, wu_ref[...].astype(jnp.bfloat16),
                preferred_element_type=jnp.float32)
    a = g * (u * jax.nn.sigmoid(u))
    part = jnp.dot(a.astype(jnp.bfloat16), wd_ref[...].astype(jnp.bfloat16),
                   preferred_element_type=jnp.float32)

    @pl.when(kk == 0)
    def _():
        out_ref[pl.ds(row0, HR), :] = part

    @pl.when(kk > 0)
    def _():
        out_ref[pl.ds(row0, HR), :] += part

    for t in range(7):
        @pl.when(k == NK - 1 + t)
        def _(t=t):
            _ar_phase(t, 0, out_ref, x16_ref, bufs, sems, i)

    @pl.when(k == 2 * NK - 1)
    def _():
        for t in range(7):
            _ar_phase(t, HR, out_ref, x16_ref, bufs, sems, i)


def kernel(x, Wg, Wu, Wd):
    group_bufs = [
        pltpu.VMEM((S3, GCOLS), jnp.bfloat16),
        pltpu.VMEM((S1, GCOLS), jnp.bfloat16),
        pltpu.VMEM((S2, GCOLS), jnp.bfloat16),
        pltpu.VMEM((S3, GCOLS), jnp.bfloat16),
        pltpu.VMEM((S2, GCOLS), jnp.float32),
    ]
    sems = pltpu.SemaphoreType.DMA((3,))
    return pl.pallas_call(
        _fused_body,
        grid=(2 * NK,),
        in_specs=[
            pl.BlockSpec(memory_space=pltpu.VMEM),
            pl.BlockSpec((D, BH), lambda k: (0, lax.rem(k, NK))),
            pl.BlockSpec((D, BH), lambda k: (0, lax.rem(k, NK))),
            pl.BlockSpec((BH, D), lambda k: (lax.rem(k, NK), 0)),
        ],
        out_specs=pl.BlockSpec(memory_space=pltpu.VMEM),
        out_shape=jax.ShapeDtypeStruct((M, D), jnp.float32),
        scratch_shapes=(
            [pltpu.VMEM((M, D), jnp.bfloat16)]
            + group_bufs * NG
            + [sems] * (4 * NG)
        ),
        compiler_params=pltpu.CompilerParams(
            collective_id=0, vmem_limit_bytes=63 * 1024 * 1024),
    )(x, Wg, Wu, Wd)
